# Initial kernel scaffold; baseline (speedup 1.0000x reference)
import functools

import jax
import jax.numpy as jnp
from jax import lax
from jax.experimental import pallas as pl
from jax.experimental.pallas import tpu as pltpu
from jax.experimental.pallas import tpu_sc as plsc

BATCH = 4096
SEQ = 200
VOCAB = 1000
BINF = 64

NUM_CORES = 2
NUM_SUBCORES = 16
NUM_WORKERS = NUM_CORES * NUM_SUBCORES

TOTAL = BATCH * SEQ
PER_WORKER = TOTAL // NUM_WORKERS
CHUNK = 512
NUM_CHUNKS = PER_WORKER // CHUNK


def _sc_gather(table, idx):
    mesh = plsc.VectorSubcoreMesh(core_axis_name="c", subcore_axis_name="s")

    @functools.partial(
        pl.kernel,
        mesh=mesh,
        out_type=jax.ShapeDtypeStruct((TOTAL, BINF), jnp.float32),
        scratch_types=[
            pltpu.VMEM((CHUNK,), jnp.int32),
            pltpu.VMEM((CHUNK, BINF), jnp.float32),
            pltpu.SemaphoreType.DMA,
        ],
    )
    def body(table_hbm, idx_hbm, out_hbm, idx_v, rows_v, sem):
        wid = lax.axis_index("s") * NUM_CORES + lax.axis_index("c")
        base = wid * PER_WORKER

        def chunk_body(g, carry):
            cbase = pl.multiple_of(base + g * CHUNK, CHUNK)
            pltpu.sync_copy(idx_hbm.at[pl.ds(cbase, CHUNK)], idx_v)
            pltpu.async_copy(table_hbm.at[idx_v], rows_v, sem).wait()
            pltpu.sync_copy(rows_v, out_hbm.at[pl.ds(cbase, CHUNK)])
            return carry

        lax.fori_loop(0, NUM_CHUNKS, chunk_body, 0)

    return body(table, idx)


def kernel(x, mapping):
    table = mapping.T
    idx = x.reshape(-1).astype(jnp.int32)
    out = _sc_gather(table, idx)
    return out.reshape(BATCH, SEQ, BINF)


# speedup vs baseline: 3.5851x; 3.5851x over previous
import functools

import jax
import jax.numpy as jnp
from jax import lax
from jax.experimental import pallas as pl
from jax.experimental.pallas import tpu as pltpu
from jax.experimental.pallas import tpu_sc as plsc

BATCH = 4096
SEQ = 200
VOCAB = 1000
BINF = 64

NUM_CORES = 2
NUM_SUBCORES = 16
NUM_WORKERS = NUM_CORES * NUM_SUBCORES

TOTAL = BATCH * SEQ
PER_WORKER = TOTAL // NUM_WORKERS
CHUNK = 512
NUM_CHUNKS = PER_WORKER // CHUNK


def _sc_gather(table, idx):
    mesh = plsc.VectorSubcoreMesh(core_axis_name="c", subcore_axis_name="s")

    @functools.partial(
        pl.kernel,
        mesh=mesh,
        compiler_params=pltpu.CompilerParams(use_tc_tiling_on_sc=False),
        out_type=jax.ShapeDtypeStruct((TOTAL, BINF), jnp.float32),
        scratch_types=[
            pltpu.VMEM((CHUNK,), jnp.int32),
            pltpu.VMEM((CHUNK, BINF), jnp.float32),
            pltpu.SemaphoreType.DMA,
        ],
    )
    def body(table_hbm, idx_hbm, out_hbm, idx_v, rows_v, sem):
        wid = lax.axis_index("s") * NUM_CORES + lax.axis_index("c")
        base = wid * PER_WORKER

        def chunk_body(g, carry):
            cbase = pl.multiple_of(base + g * CHUNK, CHUNK)
            pltpu.sync_copy(idx_hbm.at[pl.ds(cbase, CHUNK)], idx_v)
            pltpu.async_copy(table_hbm.at[idx_v], rows_v, sem).wait()
            pltpu.sync_copy(rows_v, out_hbm.at[pl.ds(cbase, CHUNK)])
            return carry

        lax.fori_loop(0, NUM_CHUNKS, chunk_body, 0)

    return body(table, idx)


def kernel(x, mapping):
    table = mapping.T
    idx = x.reshape(-1).astype(jnp.int32)
    out = _sc_gather(table, idx)
    return out.reshape(BATCH, SEQ, BINF)
